# baseline (speedup 1.0000x reference)
import jax
import jax.numpy as jnp
from jax.experimental import pallas as pl
from jax.experimental.pallas import tpu as pltpu

_T = 32768
_C = 256
_BT = 2048


def _agg_kernel(x1_ref, x2_ref, w1_ref, w2_ref, b_ref, o_ref):
    acc = jnp.dot(x1_ref[...], w1_ref[...], preferred_element_type=jnp.float32)
    acc = acc + jnp.dot(x2_ref[...], w2_ref[...], preferred_element_type=jnp.float32)
    o_ref[...] = acc + b_ref[...]


def kernel(x_1, x_2, W, b):
    W1 = W[:_C]
    W2 = W[_C:]
    b2d = b.reshape(1, _C)
    return pl.pallas_call(
        _agg_kernel,
        grid=(_T // _BT,),
        in_specs=[
            pl.BlockSpec((_BT, _C), lambda i: (i, 0)),
            pl.BlockSpec((_BT, _C), lambda i: (i, 0)),
            pl.BlockSpec((_C, _C), lambda i: (0, 0)),
            pl.BlockSpec((_C, _C), lambda i: (0, 0)),
            pl.BlockSpec((1, _C), lambda i: (0, 0)),
        ],
        out_specs=pl.BlockSpec((_BT, _C), lambda i: (i, 0)),
        out_shape=jax.ShapeDtypeStruct((_T, _C), jnp.float32),
        compiler_params=pltpu.CompilerParams(
            dimension_semantics=("parallel",),
        ),
    )(x_1, x_2, W1, W2, b2d)


# speedup vs baseline: 1.0742x; 1.0742x over previous
import jax
import jax.numpy as jnp
from jax.experimental import pallas as pl
from jax.experimental.pallas import tpu as pltpu

_T = 32768
_C = 256
_BT = 8192


def _agg_kernel(x1_ref, x2_ref, w1_ref, w2_ref, b_ref, o_ref):
    acc = jnp.dot(x1_ref[...], w1_ref[...], preferred_element_type=jnp.float32)
    acc = acc + jnp.dot(x2_ref[...], w2_ref[...], preferred_element_type=jnp.float32)
    o_ref[...] = acc + b_ref[...]


def kernel(x_1, x_2, W, b):
    W1 = W[:_C]
    W2 = W[_C:]
    b2d = b.reshape(1, _C)
    return pl.pallas_call(
        _agg_kernel,
        grid=(_T // _BT,),
        in_specs=[
            pl.BlockSpec((_BT, _C), lambda i: (i, 0)),
            pl.BlockSpec((_BT, _C), lambda i: (i, 0)),
            pl.BlockSpec((_C, _C), lambda i: (0, 0)),
            pl.BlockSpec((_C, _C), lambda i: (0, 0)),
            pl.BlockSpec((1, _C), lambda i: (0, 0)),
        ],
        out_specs=pl.BlockSpec((_BT, _C), lambda i: (i, 0)),
        out_shape=jax.ShapeDtypeStruct((_T, _C), jnp.float32),
        compiler_params=pltpu.CompilerParams(
            dimension_semantics=("parallel",),
        ),
    )(x_1, x_2, W1, W2, b2d)


# speedup vs baseline: 1.1034x; 1.0272x over previous
import jax
import jax.numpy as jnp
from jax.experimental import pallas as pl
from jax.experimental.pallas import tpu as pltpu

_T = 32768
_C = 256
_CHUNK = 1024
_NBUF = 8
_NCHUNK = _T // _CHUNK


def _agg_kernel(x1_hbm, x2_hbm, w_hbm, b_hbm, o_hbm,
                x1_buf, x2_buf, o_buf, w_buf, b_buf, in_sems, out_sems, w_sem):
    def in_copies(i, slot):
        c1 = pltpu.make_async_copy(
            x1_hbm.at[pl.ds(i * _CHUNK, _CHUNK)], x1_buf.at[slot],
            in_sems.at[slot, 0])
        c2 = pltpu.make_async_copy(
            x2_hbm.at[pl.ds(i * _CHUNK, _CHUNK)], x2_buf.at[slot],
            in_sems.at[slot, 1])
        return c1, c2

    def out_copy(i, slot):
        return pltpu.make_async_copy(
            o_buf.at[slot], o_hbm.at[pl.ds(i * _CHUNK, _CHUNK)],
            out_sems.at[slot])

    for i in range(_NBUF):
        for c in in_copies(i, i):
            c.start()
    w_copy = pltpu.make_async_copy(w_hbm, w_buf, w_sem)
    b_copy = pltpu.make_async_copy(b_hbm, b_buf, w_sem)
    w_copy.start()
    b_copy.start()
    w_copy.wait()
    b_copy.wait()

    w1 = w_buf[:_C, :]
    w2 = w_buf[_C:, :]
    bias = b_buf[...].reshape(1, _C)

    for i in range(_NCHUNK):
        slot = i % _NBUF
        c1, c2 = in_copies(i, slot)
        c1.wait()
        c2.wait()
        if i >= _NBUF:
            out_copy(i - _NBUF, slot).wait()
        acc = jnp.dot(x1_buf[slot], w1, preferred_element_type=jnp.float32)
        acc = acc + jnp.dot(x2_buf[slot], w2, preferred_element_type=jnp.float32)
        o_buf[slot] = acc + bias
        out_copy(i, slot).start()
        nxt = i + _NBUF
        if nxt < _NCHUNK:
            for c in in_copies(nxt, slot):
                c.start()

    for i in range(_NCHUNK - _NBUF, _NCHUNK):
        out_copy(i, i % _NBUF).wait()


def kernel(x_1, x_2, W, b):
    return pl.pallas_call(
        _agg_kernel,
        in_specs=[
            pl.BlockSpec(memory_space=pl.ANY),
            pl.BlockSpec(memory_space=pl.ANY),
            pl.BlockSpec(memory_space=pl.ANY),
            pl.BlockSpec(memory_space=pl.ANY),
        ],
        out_specs=pl.BlockSpec(memory_space=pl.ANY),
        out_shape=jax.ShapeDtypeStruct((_T, _C), jnp.float32),
        scratch_shapes=[
            pltpu.VMEM((_NBUF, _CHUNK, _C), jnp.float32),
            pltpu.VMEM((_NBUF, _CHUNK, _C), jnp.float32),
            pltpu.VMEM((_NBUF, _CHUNK, _C), jnp.float32),
            pltpu.VMEM((2 * _C, _C), jnp.float32),
            pltpu.VMEM((_C,), jnp.float32),
            pltpu.SemaphoreType.DMA((_NBUF, 2)),
            pltpu.SemaphoreType.DMA((_NBUF,)),
            pltpu.SemaphoreType.DMA,
        ],
    )(x_1, x_2, W, b)


# speedup vs baseline: 1.1116x; 1.0074x over previous
import jax
import jax.numpy as jnp
from jax.experimental import pallas as pl
from jax.experimental.pallas import tpu as pltpu

_T = 32768
_C = 256
_CHUNK = 1024
_NBUF = 8
_NCHUNK = _T // _CHUNK


def _agg_kernel(x1_hbm, x2_hbm, w_hbm, b_hbm, o_hbm,
                x1_buf, x2_buf, o_buf, w_buf, b_buf, in_sems, out_sems, w_sem):
    def in_copies(i, slot):
        c1 = pltpu.make_async_copy(
            x1_hbm.at[pl.ds(i * _CHUNK, _CHUNK)], x1_buf.at[slot],
            in_sems.at[slot, 0])
        c2 = pltpu.make_async_copy(
            x2_hbm.at[pl.ds(i * _CHUNK, _CHUNK)], x2_buf.at[slot],
            in_sems.at[slot, 1])
        return c1, c2

    def out_copy(i, slot):
        return pltpu.make_async_copy(
            o_buf.at[slot], o_hbm.at[pl.ds(i * _CHUNK, _CHUNK)],
            out_sems.at[slot])

    w_copy = pltpu.make_async_copy(w_hbm, w_buf, w_sem)
    b_copy = pltpu.make_async_copy(b_hbm, b_buf, w_sem)
    w_copy.start()
    b_copy.start()
    for i in range(_NBUF):
        for c in in_copies(i, i):
            c.start()
    w_copy.wait()
    b_copy.wait()

    w1 = w_buf[:_C, :]
    w2 = w_buf[_C:, :]
    bias = b_buf[...].reshape(1, _C)

    for i in range(_NCHUNK):
        slot = i % _NBUF
        c1, c2 = in_copies(i, slot)
        c1.wait()
        c2.wait()
        if i >= _NBUF:
            out_copy(i - _NBUF, slot).wait()
        acc = jnp.dot(x1_buf[slot], w1, preferred_element_type=jnp.float32)
        acc = acc + jnp.dot(x2_buf[slot], w2, preferred_element_type=jnp.float32)
        o_buf[slot] = acc + bias
        out_copy(i, slot).start()
        nxt = i + _NBUF
        if nxt < _NCHUNK:
            for c in in_copies(nxt, slot):
                c.start()

    for i in range(_NCHUNK - _NBUF, _NCHUNK):
        out_copy(i, i % _NBUF).wait()


def kernel(x_1, x_2, W, b):
    return pl.pallas_call(
        _agg_kernel,
        in_specs=[
            pl.BlockSpec(memory_space=pl.ANY),
            pl.BlockSpec(memory_space=pl.ANY),
            pl.BlockSpec(memory_space=pl.ANY),
            pl.BlockSpec(memory_space=pl.ANY),
        ],
        out_specs=pl.BlockSpec(memory_space=pl.ANY),
        out_shape=jax.ShapeDtypeStruct((_T, _C), jnp.float32),
        scratch_shapes=[
            pltpu.VMEM((_NBUF, _CHUNK, _C), jnp.float32),
            pltpu.VMEM((_NBUF, _CHUNK, _C), jnp.float32),
            pltpu.VMEM((_NBUF, _CHUNK, _C), jnp.float32),
            pltpu.VMEM((2 * _C, _C), jnp.float32),
            pltpu.VMEM((_C,), jnp.float32),
            pltpu.SemaphoreType.DMA((_NBUF, 2)),
            pltpu.SemaphoreType.DMA((_NBUF,)),
            pltpu.SemaphoreType.DMA,
        ],
    )(x_1, x_2, W, b)


# speedup vs baseline: 1.1409x; 1.0264x over previous
import jax
import jax.numpy as jnp
from jax.experimental import pallas as pl
from jax.experimental.pallas import tpu as pltpu

_T = 32768
_C = 256
_CHUNK = 1024
_NBUF = 8

_SCHEDULE = []
_sizes = [256, 256, 512] + [1024] * 30 + [512, 256, 256]
assert sum(_sizes) == _T
_off = 0
for _s in _sizes:
    _SCHEDULE.append((_off, _s))
    _off += _s


def _agg_kernel(x1_hbm, x2_hbm, w_ref, b_ref, o_hbm,
                x1_buf, x2_buf, o_buf, in_sems, out_sems):
    def in_copies(idx, slot):
        base, rows = _SCHEDULE[idx]
        c1 = pltpu.make_async_copy(
            x1_hbm.at[pl.ds(base, rows)], x1_buf.at[slot, pl.ds(0, rows)],
            in_sems.at[slot, 0])
        c2 = pltpu.make_async_copy(
            x2_hbm.at[pl.ds(base, rows)], x2_buf.at[slot, pl.ds(0, rows)],
            in_sems.at[slot, 1])
        return c1, c2

    def out_copy(idx, slot):
        base, rows = _SCHEDULE[idx]
        return pltpu.make_async_copy(
            o_buf.at[slot, pl.ds(0, rows)], o_hbm.at[pl.ds(base, rows)],
            out_sems.at[slot])

    for idx in range(_NBUF):
        for c in in_copies(idx, idx):
            c.start()

    w1 = w_ref[:_C, :]
    w2 = w_ref[_C:, :]
    bias = b_ref[...].reshape(1, _C)

    n = len(_SCHEDULE)
    for idx in range(n):
        slot = idx % _NBUF
        rows = _SCHEDULE[idx][1]
        c1, c2 = in_copies(idx, slot)
        c1.wait()
        c2.wait()
        if idx >= _NBUF:
            out_copy(idx - _NBUF, slot).wait()
        acc = jnp.dot(x1_buf[slot, :rows], w1, preferred_element_type=jnp.float32)
        acc = acc + jnp.dot(x2_buf[slot, :rows], w2,
                            preferred_element_type=jnp.float32)
        o_buf[slot, :rows] = acc + bias
        out_copy(idx, slot).start()
        nxt = idx + _NBUF
        if nxt < n:
            for c in in_copies(nxt, slot):
                c.start()

    for idx in range(n - _NBUF, n):
        out_copy(idx, idx % _NBUF).wait()


def kernel(x_1, x_2, W, b):
    return pl.pallas_call(
        _agg_kernel,
        in_specs=[
            pl.BlockSpec(memory_space=pl.ANY),
            pl.BlockSpec(memory_space=pl.ANY),
            pl.BlockSpec(memory_space=pltpu.VMEM),
            pl.BlockSpec(memory_space=pltpu.VMEM),
        ],
        out_specs=pl.BlockSpec(memory_space=pl.ANY),
        out_shape=jax.ShapeDtypeStruct((_T, _C), jnp.float32),
        scratch_shapes=[
            pltpu.VMEM((_NBUF, _CHUNK, _C), jnp.float32),
            pltpu.VMEM((_NBUF, _CHUNK, _C), jnp.float32),
            pltpu.VMEM((_NBUF, _CHUNK, _C), jnp.float32),
            pltpu.SemaphoreType.DMA((_NBUF, 2)),
            pltpu.SemaphoreType.DMA((_NBUF,)),
        ],
    )(x_1, x_2, W, b)
